# baseline (speedup 1.0000x reference)
import jax
import jax.numpy as jnp
from jax.experimental import pallas as pl
from jax.experimental.pallas import tpu as pltpu

DIM = 768
NUM_EXPERTS = 8
NUM_MICROS = 16
TOP_K = 2
TOP_K_MICROS = 8
EXPERT_DIM = 1536
MICRO_HID = DIM // 2
SEQ = 2048
TILE = 512
NUM_TILES = SEQ // TILE
EG = 2
MG = 4
E_STEPS = NUM_EXPERTS // EG
M_STEPS = NUM_MICROS // MG
EPS = 1e-5


def _gelu(v):
    return 0.5 * v * (1.0 + jax.lax.erf(v * 0.7071067811865476))


def _layer_norm(v, g, b):
    mu = jnp.mean(v, axis=-1, keepdims=True)
    var = jnp.mean((v - mu) ** 2, axis=-1, keepdims=True)
    return (v - mu) * jax.lax.rsqrt(var + EPS) * g + b


def _topk_mask_combine(probs, k):
    work = probs
    thr = None
    sel_sum = jnp.zeros(probs.shape[:-1] + (1,), probs.dtype)
    for _ in range(k):
        thr = jnp.max(work, axis=-1, keepdims=True)
        sel_sum = sel_sum + thr
        work = jnp.where(work >= thr, -jnp.inf, work)
    mask = probs >= thr
    return jnp.where(mask, probs, 0.0) / (sel_sum + 1e-8)


def _col(combine, idx):
    lane = jax.lax.broadcasted_iota(jnp.int32, combine.shape, 1)
    return jnp.sum(jnp.where(lane == idx, combine, 0.0), axis=-1,
                   keepdims=True)


def _expert_kernel(x_ref, rw_ref, rb_ref, w1_ref, b1_ref, w2_ref, b2_ref,
                   out_ref, cmb_ref):
    g = pl.program_id(0)
    t = pl.program_id(1)
    xt = x_ref[pl.ds(t * TILE, TILE), :]

    @pl.when(g == 0)
    def _router():
        logits = jnp.dot(xt, rw_ref[...], preferred_element_type=jnp.float32)
        logits = logits + rb_ref[...]
        probs = jax.nn.softmax(logits, axis=-1)
        cmb_ref[pl.ds(t * TILE, TILE), :] = _topk_mask_combine(probs, TOP_K)

    combine = cmb_ref[pl.ds(t * TILE, TILE), :]

    h = jnp.dot(xt, w1_ref[0], preferred_element_type=jnp.float32)
    h = _gelu(h + b1_ref[0])
    acc = None
    for j in range(EG):
        eo = jnp.dot(h[:, j * EXPERT_DIM:(j + 1) * EXPERT_DIM], w2_ref[0, j],
                     preferred_element_type=jnp.float32)
        eo = (eo + b2_ref[0, j]) * _col(combine, g * EG + j)
        acc = eo if acc is None else acc + eo

    @pl.when(g == 0)
    def _init():
        out_ref[pl.ds(t * TILE, TILE), :] = acc

    @pl.when(g > 0)
    def _acc():
        out_ref[pl.ds(t * TILE, TILE), :] += acc


def _micro_kernel(eo_ref, rw_ref, rb_ref, w1_ref, b1_ref, w2_ref, b2_ref,
                  lng_ref, lnb_ref, ng_ref, nb_ref, out_ref, acc_ref,
                  cmb_ref):
    g = pl.program_id(0)
    t = pl.program_id(1)
    xt = eo_ref[pl.ds(t * TILE, TILE), :]

    @pl.when(g == 0)
    def _router():
        logits = jnp.dot(xt, rw_ref[...], preferred_element_type=jnp.float32)
        logits = logits + rb_ref[...]
        probs = jax.nn.softmax(logits, axis=-1)
        cmb_ref[pl.ds(t * TILE, TILE), :] = _topk_mask_combine(
            probs, TOP_K_MICROS)

    mcombine = cmb_ref[pl.ds(t * TILE, TILE), :]

    mh = jnp.dot(xt, w1_ref[0], preferred_element_type=jnp.float32)
    mh = _gelu(mh + b1_ref[0])
    acc = None
    for j in range(MG):
        mf = jnp.dot(mh[:, j * MICRO_HID:(j + 1) * MICRO_HID], w2_ref[0, j],
                     preferred_element_type=jnp.float32)
        pre = xt + mf + b2_ref[0, j]
        mo = _layer_norm(pre, lng_ref[0, j], lnb_ref[0, j])
        mo = mo * _col(mcombine, g * MG + j)
        acc = mo if acc is None else acc + mo

    @pl.when(g == 0)
    def _init():
        acc_ref[pl.ds(t * TILE, TILE), :] = acc

    @pl.when(g > 0)
    def _acc():
        acc_ref[pl.ds(t * TILE, TILE), :] += acc

    @pl.when(g == M_STEPS - 1)
    def _final():
        combined = xt + 0.1 * acc_ref[pl.ds(t * TILE, TILE), :]
        out_ref[pl.ds(t * TILE, TILE), :] = _layer_norm(
            combined, ng_ref[...], nb_ref[...])


def _resident(shape):
    return pl.BlockSpec(shape, lambda *_: tuple(0 for _ in shape))


def _per_g(shape):
    return pl.BlockSpec(shape, lambda g, t: (g,) + tuple(0 for _ in shape[1:]))


def _group_w1(w, groups, per, d_in, d_hid):
    return w.reshape(groups, per, d_in, d_hid).transpose(0, 2, 1, 3).reshape(
        groups, d_in, per * d_hid)


@jax.jit
def kernel(x, router_W, router_b, expert_W1, expert_b1, expert_W2, expert_b2,
           micro_router_W, micro_router_b, micro_W1, micro_b1, micro_W2,
           micro_b2, micro_ln_g, micro_ln_b, norm_g, norm_b):
    B, S, D = x.shape
    xf = x.reshape(S, D)

    expert_output = pl.pallas_call(
        _expert_kernel,
        grid=(E_STEPS, NUM_TILES),
        in_specs=[
            _resident((S, D)),
            _resident((D, NUM_EXPERTS)),
            _resident((1, NUM_EXPERTS)),
            _per_g((1, DIM, EG * EXPERT_DIM)),
            _per_g((1, 1, EG * EXPERT_DIM)),
            _per_g((1, EG, EXPERT_DIM, DIM)),
            _per_g((1, EG, DIM)),
        ],
        out_specs=_resident((S, D)),
        out_shape=jax.ShapeDtypeStruct((S, D), jnp.float32),
        scratch_shapes=[pltpu.VMEM((S, NUM_EXPERTS), jnp.float32)],
        compiler_params=pltpu.CompilerParams(
            dimension_semantics=("arbitrary", "arbitrary"),
        ),
    )(xf, router_W, router_b.reshape(1, -1),
      _group_w1(expert_W1, E_STEPS, EG, DIM, EXPERT_DIM),
      expert_b1.reshape(E_STEPS, 1, EG * EXPERT_DIM),
      expert_W2.reshape(E_STEPS, EG, EXPERT_DIM, DIM),
      expert_b2.reshape(E_STEPS, EG, DIM))

    out = pl.pallas_call(
        _micro_kernel,
        grid=(M_STEPS, NUM_TILES),
        in_specs=[
            _resident((S, D)),
            _resident((D, NUM_MICROS)),
            _resident((1, NUM_MICROS)),
            _per_g((1, DIM, MG * MICRO_HID)),
            _per_g((1, 1, MG * MICRO_HID)),
            _per_g((1, MG, MICRO_HID, DIM)),
            _per_g((1, MG, DIM)),
            _per_g((1, MG, DIM)),
            _per_g((1, MG, DIM)),
            _resident((1, DIM)),
            _resident((1, DIM)),
        ],
        out_specs=_resident((S, D)),
        out_shape=jax.ShapeDtypeStruct((S, D), jnp.float32),
        scratch_shapes=[pltpu.VMEM((S, D), jnp.float32),
                        pltpu.VMEM((S, NUM_MICROS), jnp.float32)],
        compiler_params=pltpu.CompilerParams(
            dimension_semantics=("arbitrary", "arbitrary"),
        ),
    )(expert_output, micro_router_W, micro_router_b.reshape(1, -1),
      _group_w1(micro_W1, M_STEPS, MG, DIM, MICRO_HID),
      micro_b1.reshape(M_STEPS, 1, MG * MICRO_HID),
      micro_W2.reshape(M_STEPS, MG, MICRO_HID, DIM),
      micro_b2.reshape(M_STEPS, MG, DIM),
      micro_ln_g.reshape(M_STEPS, MG, DIM),
      micro_ln_b.reshape(M_STEPS, MG, DIM),
      norm_g.reshape(1, -1), norm_b.reshape(1, -1))

    return out.reshape(B, S, D)


# speedup vs baseline: 1.7711x; 1.7711x over previous
import jax
import jax.numpy as jnp
from jax.experimental import pallas as pl
from jax.experimental.pallas import tpu as pltpu

DIM = 768
NUM_EXPERTS = 8
NUM_MICROS = 16
TOP_K = 2
TOP_K_MICROS = 8
EXPERT_DIM = 1536
MICRO_HID = DIM // 2
SEQ = 2048
TILE = 512
NUM_TILES = SEQ // TILE
EG = 2
MG = 4
E_STEPS = NUM_EXPERTS // EG
M_STEPS = NUM_MICROS // MG
EPS = 1e-5


def _gelu(v):
    return 0.5 * v * (1.0 + jax.lax.erf(v * 0.7071067811865476))


def _layer_norm(v, g, b):
    mu = jnp.mean(v, axis=-1, keepdims=True)
    var = jnp.mean((v - mu) ** 2, axis=-1, keepdims=True)
    return (v - mu) * jax.lax.rsqrt(var + EPS) * g + b


def _topk_mask_combine(probs, k):
    work = probs
    thr = None
    sel_sum = jnp.zeros(probs.shape[:-1] + (1,), probs.dtype)
    for _ in range(k):
        thr = jnp.max(work, axis=-1, keepdims=True)
        sel_sum = sel_sum + thr
        work = jnp.where(work >= thr, -jnp.inf, work)
    mask = probs >= thr
    return jnp.where(mask, probs, 0.0) / (sel_sum + 1e-8)


def _col(combine, idx):
    lane = jax.lax.broadcasted_iota(jnp.int32, combine.shape, 1)
    return jnp.sum(jnp.where(lane == idx, combine, 0.0), axis=-1,
                   keepdims=True)


def _expert_kernel(x_ref, rw_ref, rb_ref, w1_ref, b1_ref, w2_ref, b2_ref,
                   out_ref, cmb_ref):
    g = pl.program_id(0)
    t = pl.program_id(1)
    xt = x_ref[pl.ds(t * TILE, TILE), :]

    @pl.when(g == 0)
    def _router():
        logits = jnp.dot(xt, rw_ref[...], preferred_element_type=jnp.float32)
        logits = logits + rb_ref[...]
        probs = jax.nn.softmax(logits, axis=-1)
        cmb_ref[pl.ds(t * TILE, TILE), :] = _topk_mask_combine(probs, TOP_K)

    combine = cmb_ref[pl.ds(t * TILE, TILE), :]

    acc = None
    for j in range(EG):
        h = jnp.dot(xt, w1_ref[0, j], preferred_element_type=jnp.float32)
        h = _gelu(h + b1_ref[0, j])
        eo = jnp.dot(h, w2_ref[0, j], preferred_element_type=jnp.float32)
        eo = (eo + b2_ref[0, j]) * _col(combine, g * EG + j)
        acc = eo if acc is None else acc + eo

    @pl.when(g == 0)
    def _init():
        out_ref[pl.ds(t * TILE, TILE), :] = acc

    @pl.when(g > 0)
    def _acc():
        out_ref[pl.ds(t * TILE, TILE), :] += acc


def _micro_kernel(eo_ref, rw_ref, rb_ref, w1_ref, b1_ref, w2_ref, b2_ref,
                  lng_ref, lnb_ref, ng_ref, nb_ref, out_ref, acc_ref,
                  cmb_ref):
    g = pl.program_id(0)
    t = pl.program_id(1)
    xt = eo_ref[pl.ds(t * TILE, TILE), :]

    @pl.when(g == 0)
    def _router():
        logits = jnp.dot(xt, rw_ref[...], preferred_element_type=jnp.float32)
        logits = logits + rb_ref[...]
        probs = jax.nn.softmax(logits, axis=-1)
        cmb_ref[pl.ds(t * TILE, TILE), :] = _topk_mask_combine(
            probs, TOP_K_MICROS)

    mcombine = cmb_ref[pl.ds(t * TILE, TILE), :]

    acc = None
    for j in range(MG):
        mh = jnp.dot(xt, w1_ref[0, j], preferred_element_type=jnp.float32)
        mh = _gelu(mh + b1_ref[0, j])
        mf = jnp.dot(mh, w2_ref[0, j], preferred_element_type=jnp.float32)
        pre = xt + mf + b2_ref[0, j]
        mo = _layer_norm(pre, lng_ref[0, j], lnb_ref[0, j])
        mo = mo * _col(mcombine, g * MG + j)
        acc = mo if acc is None else acc + mo

    @pl.when(g == 0)
    def _init():
        acc_ref[pl.ds(t * TILE, TILE), :] = acc

    @pl.when(g > 0)
    def _acc():
        acc_ref[pl.ds(t * TILE, TILE), :] += acc

    @pl.when(g == M_STEPS - 1)
    def _final():
        combined = xt + 0.1 * acc_ref[pl.ds(t * TILE, TILE), :]
        out_ref[pl.ds(t * TILE, TILE), :] = _layer_norm(
            combined, ng_ref[...], nb_ref[...])


def _resident(shape):
    return pl.BlockSpec(shape, lambda *_: tuple(0 for _ in shape))


def _per_g(shape):
    return pl.BlockSpec(shape, lambda g, t: (g,) + tuple(0 for _ in shape[1:]))


@jax.jit
def kernel(x, router_W, router_b, expert_W1, expert_b1, expert_W2, expert_b2,
           micro_router_W, micro_router_b, micro_W1, micro_b1, micro_W2,
           micro_b2, micro_ln_g, micro_ln_b, norm_g, norm_b):
    B, S, D = x.shape
    xf = x.reshape(S, D)

    expert_output = pl.pallas_call(
        _expert_kernel,
        grid=(E_STEPS, NUM_TILES),
        in_specs=[
            _resident((S, D)),
            _resident((D, NUM_EXPERTS)),
            _resident((1, NUM_EXPERTS)),
            _per_g((1, EG, DIM, EXPERT_DIM)),
            _per_g((1, EG, 1, EXPERT_DIM)),
            _per_g((1, EG, EXPERT_DIM, DIM)),
            _per_g((1, EG, DIM)),
        ],
        out_specs=_resident((S, D)),
        out_shape=jax.ShapeDtypeStruct((S, D), jnp.float32),
        scratch_shapes=[pltpu.VMEM((S, NUM_EXPERTS), jnp.float32)],
        compiler_params=pltpu.CompilerParams(
            dimension_semantics=("arbitrary", "arbitrary"),
        ),
    )(xf, router_W, router_b.reshape(1, -1),
      expert_W1.reshape(E_STEPS, EG, DIM, EXPERT_DIM),
      expert_b1.reshape(E_STEPS, EG, 1, EXPERT_DIM),
      expert_W2.reshape(E_STEPS, EG, EXPERT_DIM, DIM),
      expert_b2.reshape(E_STEPS, EG, DIM))

    out = pl.pallas_call(
        _micro_kernel,
        grid=(M_STEPS, NUM_TILES),
        in_specs=[
            _resident((S, D)),
            _resident((D, NUM_MICROS)),
            _resident((1, NUM_MICROS)),
            _per_g((1, MG, DIM, MICRO_HID)),
            _per_g((1, MG, 1, MICRO_HID)),
            _per_g((1, MG, MICRO_HID, DIM)),
            _per_g((1, MG, DIM)),
            _per_g((1, MG, DIM)),
            _per_g((1, MG, DIM)),
            _resident((1, DIM)),
            _resident((1, DIM)),
        ],
        out_specs=_resident((S, D)),
        out_shape=jax.ShapeDtypeStruct((S, D), jnp.float32),
        scratch_shapes=[pltpu.VMEM((S, D), jnp.float32),
                        pltpu.VMEM((S, NUM_MICROS), jnp.float32)],
        compiler_params=pltpu.CompilerParams(
            dimension_semantics=("arbitrary", "arbitrary"),
        ),
    )(expert_output, micro_router_W, micro_router_b.reshape(1, -1),
      micro_W1.reshape(M_STEPS, MG, DIM, MICRO_HID),
      micro_b1.reshape(M_STEPS, MG, 1, MICRO_HID),
      micro_W2.reshape(M_STEPS, MG, MICRO_HID, DIM),
      micro_b2.reshape(M_STEPS, MG, DIM),
      micro_ln_g.reshape(M_STEPS, MG, DIM),
      micro_ln_b.reshape(M_STEPS, MG, DIM),
      norm_g.reshape(1, -1), norm_b.reshape(1, -1))

    return out.reshape(B, S, D)
